# Initial kernel scaffold; baseline (speedup 1.0000x reference)
import jax
import jax.numpy as jnp
from jax.experimental import pallas as pl


def kernel(x, edge_index, W):
    raise NotImplementedError("write your pallas kernel here")


# speedup vs baseline: 2.2213x; 2.2213x over previous
import functools

import jax
import jax.numpy as jnp
from jax import lax
from jax.experimental import pallas as pl
from jax.experimental.pallas import tpu as pltpu
from jax.experimental.pallas import tpu_sc as plsc

N = 10000
E = 320000
D = 128
K = 10

NC = 2
NS = 16
NG = 4
GW = D // NG
PASSES = NG // NC
CHUNK = 128
NCHUNK = E // CHUNK
CPT = -(-NCHUNK // NS)
ROWS = 640
NPAD = NS * ROWS
ZROWS = 128


def _sc_propagate(x4, src, dst, zeros2d, zeros1d):
    mesh = plsc.VectorSubcoreMesh(core_axis_name="c", subcore_axis_name="s")

    @functools.partial(
        pl.kernel,
        mesh=mesh,
        compiler_params=pltpu.CompilerParams(use_tc_tiling_on_sc=False),
        out_type=[
            jax.ShapeDtypeStruct((K, NG, NPAD, GW), jnp.float32),
            jax.ShapeDtypeStruct((NG * NPAD, GW), jnp.float32),
        ],
        scratch_types=[
            pltpu.VMEM_SHARED((NPAD, GW), jnp.float32),
            pltpu.VMEM_SHARED((NPAD,), jnp.float32),
            pltpu.VMEM((CHUNK,), jnp.int32),
            pltpu.VMEM((CHUNK,), jnp.int32),
            pltpu.VMEM((CHUNK,), jnp.float32),
            pltpu.VMEM((CHUNK, GW), jnp.float32),
            pltpu.VMEM((ROWS, GW), jnp.float32),
            pltpu.VMEM((ZROWS, GW), jnp.float32),
            pltpu.VMEM((ROWS,), jnp.float32),
            pltpu.VMEM((ROWS, GW), jnp.float32),
        ],
    )
    def body(x_hbm, src_hbm, dst_hbm, z2_hbm, z1_hbm, hs_hbm, g_hbm,
             acc_sh, degs_sh,
             srcbuf, dstbuf, onesbuf, rowbuf, buf, zb, degbuf, nb):
        c = lax.axis_index("c")
        s = lax.axis_index("s")
        row0 = s * ROWS

        pltpu.sync_copy(z2_hbm, zb)
        for m in range(ROWS // ZROWS):
            pltpu.sync_copy(zb, acc_sh.at[pl.ds(row0 + m * ZROWS, ZROWS)])
        pltpu.sync_copy(z1_hbm, degs_sh.at[pl.ds(row0, ROWS)])

        def fill_ones(i, _):
            onesbuf[pl.ds(i * 16, 16)] = jnp.full((16,), 1.0, jnp.float32)
            return 0
        lax.fori_loop(0, CHUNK // 16, fill_ones, 0)
        plsc.subcore_barrier()

        def deg_body(i, _):
            j = s + NS * i

            @pl.when(j < NCHUNK)
            def _():
                pltpu.sync_copy(dst_hbm.at[pl.ds(j * CHUNK, CHUNK)], dstbuf)
                pltpu.sync_copy(onesbuf, degs_sh.at[dstbuf], add=True)
            return 0
        lax.fori_loop(0, CPT, deg_body, 0)
        plsc.subcore_barrier()

        pltpu.sync_copy(degs_sh.at[pl.ds(row0, ROWS)], degbuf)

        def norm_body(i, _):
            d = degbuf[pl.ds(i * 16, 16)]
            y = 0.5 * (d + 1.0)
            for _ in range(16):
                y = 0.5 * (y + d / y)
            v = 1.0 / y
            for l in range(16):
                bc = jnp.broadcast_to(v[l], (16,))
                for jj in range(GW // 16):
                    nb[i * 16 + l, pl.ds(jj * 16, 16)] = bc
            return 0
        lax.fori_loop(0, ROWS // 16, norm_body, 0)

        def scale_rows(r, _):
            for jj in range(GW // 16):
                sl = pl.ds(jj * 16, 16)
                buf[r, sl] = buf[r, sl] * nb[r, sl]
            return 0

        for p in range(PASSES):
            grp = c * PASSES + p
            goff = grp * NPAD
            pltpu.sync_copy(x_hbm.at[grp, pl.ds(row0, ROWS)], buf)
            lax.fori_loop(0, ROWS, scale_rows, 0)
            pltpu.sync_copy(buf, g_hbm.at[pl.ds(goff + row0, ROWS)])
        plsc.subcore_barrier()

        def hop(k, _):
            for p in range(PASSES):
                grp = c * PASSES + p
                goff = grp * NPAD

                def edge_body(i, _2):
                    j = s + NS * i

                    @pl.when(j < NCHUNK)
                    def _():
                        pltpu.sync_copy(src_hbm.at[pl.ds(j * CHUNK, CHUNK)],
                                        srcbuf)
                        pltpu.sync_copy(dst_hbm.at[pl.ds(j * CHUNK, CHUNK)],
                                        dstbuf)

                        def adj(m, _3):
                            sl = pl.ds(m * 16, 16)
                            srcbuf[sl] = srcbuf[sl] + goff
                            return 0
                        lax.fori_loop(0, CHUNK // 16, adj, 0)
                        pltpu.sync_copy(g_hbm.at[srcbuf], rowbuf)
                        pltpu.sync_copy(rowbuf, acc_sh.at[dstbuf], add=True)
                    return 0
                lax.fori_loop(0, CPT, edge_body, 0)
                plsc.subcore_barrier()

                pltpu.sync_copy(acc_sh.at[pl.ds(row0, ROWS)], buf)
                lax.fori_loop(0, ROWS, scale_rows, 0)
                pltpu.sync_copy(buf, hs_hbm.at[k, grp, pl.ds(row0, ROWS)])
                lax.fori_loop(0, ROWS, scale_rows, 0)
                pltpu.sync_copy(buf, g_hbm.at[pl.ds(goff + row0, ROWS)])
                for m in range(ROWS // ZROWS):
                    pltpu.sync_copy(
                        zb, acc_sh.at[pl.ds(row0 + m * ZROWS, ZROWS)])
                plsc.subcore_barrier()
            return 0
        lax.fori_loop(0, K, hop, 0)

    return body(x4, src, dst, zeros2d, zeros1d)


RB = 1000


def _combine_body(x_ref, hs_ref, w_ref, o_ref):
    w = w_ref[...]
    xb = x_ref[...]
    sc = jnp.sum(xb * w, axis=1, keepdims=True)
    acc = xb / (1.0 + jnp.exp(-sc))
    for k in range(K):
        h = jnp.concatenate([hs_ref[k, g] for g in range(NG)], axis=1)
        sc = jnp.sum(h * w, axis=1, keepdims=True)
        acc = acc + h / (1.0 + jnp.exp(-sc))
    o_ref[...] = acc


def _combine(x, hs, W):
    return pl.pallas_call(
        _combine_body,
        grid=(N // RB,),
        in_specs=[
            pl.BlockSpec((RB, D), lambda i: (i, 0)),
            pl.BlockSpec((K, NG, RB, GW), lambda i: (0, 0, i, 0)),
            pl.BlockSpec((1, D), lambda i: (0, 0)),
        ],
        out_specs=pl.BlockSpec((RB, D), lambda i: (i, 0)),
        out_shape=jax.ShapeDtypeStruct((N, D), jnp.float32),
    )(x, hs, W)


def kernel(x, edge_index, W):
    x4 = jnp.zeros((NG, NPAD, GW), jnp.float32)
    x4 = x4.at[:, :N].set(jnp.transpose(x.reshape(N, NG, GW), (1, 0, 2)))
    zeros2d = jnp.zeros((ZROWS, GW), jnp.float32)
    zeros1d = jnp.zeros((ROWS,), jnp.float32)
    hs, _ = _sc_propagate(x4, edge_index[0], edge_index[1], zeros2d, zeros1d)
    return _combine(x, hs, W)


# speedup vs baseline: 7.5648x; 3.4056x over previous
import functools

import jax
import jax.numpy as jnp
from jax import lax
from jax.experimental import pallas as pl
from jax.experimental.pallas import tpu as pltpu
from jax.experimental.pallas import tpu_sc as plsc

N = 10000
E = 320000
D = 128
K = 10

NC = 2
NS = 16
NG = 4
GW = D // NG
PASSES = NG // NC
CHUNK = 128
ROWS = 640
NPAD = NS * ROWS
ZROWS = 128
NBUF = 8
CPT = 160
NGRP = CPT // NBUF
E2 = NS * CPT * CHUNK
PADC = E2 - E


def _sc_propagate(x4, srcflat, dstp, zeros2d, zeros1d):
    mesh = plsc.VectorSubcoreMesh(core_axis_name="c", subcore_axis_name="s")

    @functools.partial(
        pl.kernel,
        mesh=mesh,
        compiler_params=pltpu.CompilerParams(use_tc_tiling_on_sc=False),
        out_type=[
            jax.ShapeDtypeStruct((K, NG, NPAD, GW), jnp.float32),
            jax.ShapeDtypeStruct((NG * NPAD, GW), jnp.float32),
        ],
        scratch_types=[
            pltpu.VMEM_SHARED((NPAD, GW), jnp.float32),
            pltpu.VMEM_SHARED((NPAD,), jnp.float32),
            [pltpu.VMEM((CHUNK,), jnp.int32) for _ in range(NBUF)],
            [pltpu.VMEM((CHUNK,), jnp.int32) for _ in range(NBUF)],
            [pltpu.VMEM((CHUNK, GW), jnp.float32) for _ in range(NBUF)],
            pltpu.VMEM((CHUNK,), jnp.float32),
            pltpu.VMEM((ROWS, GW), jnp.float32),
            pltpu.VMEM((ZROWS, GW), jnp.float32),
            pltpu.VMEM((ROWS,), jnp.float32),
            pltpu.VMEM((ROWS, GW), jnp.float32),
            [pltpu.SemaphoreType.DMA for _ in range(NBUF)],
            [pltpu.SemaphoreType.DMA for _ in range(NBUF)],
            [pltpu.SemaphoreType.DMA for _ in range(NBUF)],
        ],
    )
    def body(x_hbm, src_hbm, dst_hbm, z2_hbm, z1_hbm, hs_hbm, g_hbm,
             acc_sh, degs_sh, srcb, dstb, rowb,
             onesbuf, buf, zb, degbuf, nb, semI, semG, semS):
        c = lax.axis_index("c")
        s = lax.axis_index("s")
        row0 = s * ROWS
        chunk0 = s * CPT

        pltpu.sync_copy(z2_hbm, zb)
        for m in range(ROWS // ZROWS):
            pltpu.sync_copy(zb, acc_sh.at[pl.ds(row0 + m * ZROWS, ZROWS)])
        pltpu.sync_copy(z1_hbm, degs_sh.at[pl.ds(row0, ROWS)])

        def fill_ones(i, _):
            onesbuf[pl.ds(i * 16, 16)] = jnp.full((16,), 1.0, jnp.float32)
            return 0
        lax.fori_loop(0, CHUNK // 16, fill_ones, 0)
        plsc.subcore_barrier()

        def deg_group(n, _):
            for b in range(NBUF):
                e0 = (chunk0 + n * NBUF + b) * CHUNK
                pltpu.make_async_copy(
                    dst_hbm.at[pl.ds(e0, CHUNK)], dstb[b], semI[b]).start()
            for b in range(NBUF):
                pltpu.make_async_copy(
                    dst_hbm.at[pl.ds(0, CHUNK)], dstb[b], semI[b]).wait()
                pltpu.make_async_copy(
                    onesbuf, degs_sh.at[dstb[b]], semS[b]).start(add=True)
            for b in range(NBUF):
                pltpu.make_async_copy(
                    onesbuf, degs_sh.at[dstb[b]], semS[b]).wait()
            return 0
        lax.fori_loop(0, NGRP, deg_group, 0)
        plsc.subcore_barrier()

        pltpu.sync_copy(degs_sh.at[pl.ds(row0, ROWS)], degbuf)

        def norm_body(i, _):
            d = degbuf[pl.ds(i * 16, 16)]
            y = 0.5 * (d + 1.0)
            for _ in range(16):
                y = 0.5 * (y + d / y)
            v = 1.0 / y
            for l in range(16):
                bc = jnp.broadcast_to(v[l], (16,))
                for jj in range(GW // 16):
                    nb[i * 16 + l, pl.ds(jj * 16, 16)] = bc
            return 0
        lax.fori_loop(0, ROWS // 16, norm_body, 0)

        def scale_rows(r, _):
            for jj in range(GW // 16):
                sl = pl.ds(jj * 16, 16)
                buf[r, sl] = buf[r, sl] * nb[r, sl]
            return 0

        for p in range(PASSES):
            grp = c * PASSES + p
            goff = grp * NPAD
            pltpu.sync_copy(x_hbm.at[grp, pl.ds(row0, ROWS)], buf)
            lax.fori_loop(0, ROWS, scale_rows, 0)
            pltpu.sync_copy(buf, g_hbm.at[pl.ds(goff + row0, ROWS)])
        plsc.subcore_barrier()

        def hop(k, _):
            for p in range(PASSES):
                grp = c * PASSES + p
                goff = grp * NPAD
                soff = grp * E2

                def edge_group(n, _2):
                    for b in range(NBUF):
                        e0 = (chunk0 + n * NBUF + b) * CHUNK
                        pltpu.make_async_copy(
                            src_hbm.at[pl.ds(soff + e0, CHUNK)],
                            srcb[b], semI[b]).start()
                        pltpu.make_async_copy(
                            dst_hbm.at[pl.ds(e0, CHUNK)],
                            dstb[b], semI[b]).start()
                    for b in range(NBUF):
                        pltpu.make_async_copy(
                            src_hbm.at[pl.ds(0, CHUNK)],
                            srcb[b], semI[b]).wait()
                        pltpu.make_async_copy(
                            dst_hbm.at[pl.ds(0, CHUNK)],
                            dstb[b], semI[b]).wait()
                        pltpu.make_async_copy(
                            g_hbm.at[srcb[b]], rowb[b], semG[b]).start()
                    for b in range(NBUF):
                        pltpu.make_async_copy(
                            g_hbm.at[srcb[b]], rowb[b], semG[b]).wait()
                        pltpu.make_async_copy(
                            rowb[b], acc_sh.at[dstb[b]],
                            semS[b]).start(add=True)
                    for b in range(NBUF):
                        pltpu.make_async_copy(
                            rowb[b], acc_sh.at[dstb[b]], semS[b]).wait()
                    return 0
                lax.fori_loop(0, NGRP, edge_group, 0)
                plsc.subcore_barrier()

                pltpu.sync_copy(acc_sh.at[pl.ds(row0, ROWS)], buf)
                lax.fori_loop(0, ROWS, scale_rows, 0)
                pltpu.sync_copy(buf, hs_hbm.at[k, grp, pl.ds(row0, ROWS)])
                lax.fori_loop(0, ROWS, scale_rows, 0)
                pltpu.sync_copy(buf, g_hbm.at[pl.ds(goff + row0, ROWS)])
                for m in range(ROWS // ZROWS):
                    pltpu.sync_copy(
                        zb, acc_sh.at[pl.ds(row0 + m * ZROWS, ZROWS)])
                plsc.subcore_barrier()
            return 0
        lax.fori_loop(0, K, hop, 0)

    return body(x4, srcflat, dstp, zeros2d, zeros1d)


RB = 1000


def _combine_body(x_ref, hs_ref, w_ref, o_ref):
    w = w_ref[...]
    xb = x_ref[...]
    sc = jnp.sum(xb * w, axis=1, keepdims=True)
    acc = xb / (1.0 + jnp.exp(-sc))
    for k in range(K):
        h = jnp.concatenate([hs_ref[k, g] for g in range(NG)], axis=1)
        sc = jnp.sum(h * w, axis=1, keepdims=True)
        acc = acc + h / (1.0 + jnp.exp(-sc))
    o_ref[...] = acc


def _combine(x, hs, W):
    return pl.pallas_call(
        _combine_body,
        grid=(N // RB,),
        in_specs=[
            pl.BlockSpec((RB, D), lambda i: (i, 0)),
            pl.BlockSpec((K, NG, RB, GW), lambda i: (0, 0, i, 0)),
            pl.BlockSpec((1, D), lambda i: (0, 0)),
        ],
        out_specs=pl.BlockSpec((RB, D), lambda i: (i, 0)),
        out_shape=jax.ShapeDtypeStruct((N, D), jnp.float32),
    )(x, hs, W)


def kernel(x, edge_index, W):
    src = edge_index[0]
    dst = edge_index[1]
    x4 = jnp.zeros((NG, NPAD, GW), jnp.float32)
    x4 = x4.at[:, :N].set(jnp.transpose(x.reshape(N, NG, GW), (1, 0, 2)))
    pad_rows = N + jnp.arange(PADC, dtype=jnp.int32) % (NPAD - N)
    srcp = jnp.concatenate([src, pad_rows])
    dstp = jnp.concatenate([dst, pad_rows])
    offs = (jnp.arange(NG, dtype=jnp.int32) * NPAD)[:, None]
    srcflat = (srcp[None, :] + offs).reshape(-1)
    zeros2d = jnp.zeros((ZROWS, GW), jnp.float32)
    zeros1d = jnp.zeros((ROWS,), jnp.float32)
    hs, _ = _sc_propagate(x4, srcflat, dstp, zeros2d, zeros1d)
    return _combine(x, hs, W)


# speedup vs baseline: 8.7052x; 1.1507x over previous
import functools

import jax
import jax.numpy as jnp
from jax import lax
from jax.experimental import pallas as pl
from jax.experimental.pallas import tpu as pltpu
from jax.experimental.pallas import tpu_sc as plsc

N = 10000
E = 320000
D = 128
K = 10

NC = 2
NS = 16
NG = 4
GW = D // NG
PASSES = NG // NC
CHUNK = 128
ROWS = 640
NPAD = NS * ROWS
ZROWS = 128
NBUF = 10
CPT = 160
NGRP = CPT // NBUF
BLK = NBUF * CHUNK
E2 = NS * CPT * CHUNK
E3 = E2 + BLK
PADC = E3 - E


def _sc_propagate(x4, srcflat, dstp, zeros2d, zeros1d):
    mesh = plsc.VectorSubcoreMesh(core_axis_name="c", subcore_axis_name="s")

    @functools.partial(
        pl.kernel,
        mesh=mesh,
        compiler_params=pltpu.CompilerParams(use_tc_tiling_on_sc=False),
        out_type=[
            jax.ShapeDtypeStruct((K, NG, NPAD, GW), jnp.float32),
            jax.ShapeDtypeStruct((NG * NPAD, GW), jnp.float32),
        ],
        scratch_types=[
            pltpu.VMEM_SHARED((NPAD, GW), jnp.float32),
            pltpu.VMEM_SHARED((NPAD,), jnp.float32),
            [pltpu.VMEM((BLK,), jnp.int32) for _ in range(2)],
            [pltpu.VMEM((BLK,), jnp.int32) for _ in range(2)],
            [pltpu.VMEM((CHUNK, GW), jnp.float32) for _ in range(NBUF)],
            pltpu.VMEM((CHUNK,), jnp.float32),
            pltpu.VMEM((ROWS, GW), jnp.float32),
            pltpu.VMEM((ZROWS, GW), jnp.float32),
            pltpu.VMEM((ROWS,), jnp.float32),
            pltpu.VMEM((ROWS, GW), jnp.float32),
            [pltpu.SemaphoreType.DMA for _ in range(2)],
            [pltpu.SemaphoreType.DMA for _ in range(NBUF)],
            [pltpu.SemaphoreType.DMA for _ in range(NBUF)],
        ],
    )
    def body(x_hbm, src_hbm, dst_hbm, z2_hbm, z1_hbm, hs_hbm, g_hbm,
             acc_sh, degs_sh, sblk, dblk, rowb,
             onesbuf, buf, zb, degbuf, nb, semB, semG, semS):
        c = lax.axis_index("c")
        s = lax.axis_index("s")
        row0 = s * ROWS
        chunk0 = s * CPT

        pltpu.sync_copy(z2_hbm, zb)
        for m in range(ROWS // ZROWS):
            pltpu.sync_copy(zb, acc_sh.at[pl.ds(row0 + m * ZROWS, ZROWS)])
        pltpu.sync_copy(z1_hbm, degs_sh.at[pl.ds(row0, ROWS)])

        def fill_ones(i, _):
            onesbuf[pl.ds(i * 16, 16)] = jnp.full((16,), 1.0, jnp.float32)
            return 0
        lax.fori_loop(0, CHUNK // 16, fill_ones, 0)
        plsc.subcore_barrier()

        def dblk_start(w, n):
            e0 = (chunk0 + n * NBUF) * CHUNK
            pltpu.make_async_copy(
                dst_hbm.at[pl.ds(e0, BLK)], dblk[w], semB[w]).start()

        def dblk_wait(w):
            pltpu.make_async_copy(
                dst_hbm.at[pl.ds(0, BLK)], dblk[w], semB[w]).wait()

        def sblk_start(w, n, soff):
            e0 = (chunk0 + n * NBUF) * CHUNK
            pltpu.make_async_copy(
                src_hbm.at[pl.ds(soff + e0, BLK)], sblk[w], semB[w]).start()

        def sblk_wait(w):
            pltpu.make_async_copy(
                src_hbm.at[pl.ds(0, BLK)], sblk[w], semB[w]).wait()

        def didx(w, b):
            return dblk[w].at[pl.ds(b * CHUNK, CHUNK)]

        def sidx(w, b):
            return sblk[w].at[pl.ds(b * CHUNK, CHUNK)]

        def deg_proc(w):
            for b in range(NBUF):
                pltpu.make_async_copy(
                    onesbuf, degs_sh.at[didx(w, b)], semS[b]).start(add=True)
            for b in range(NBUF):
                pltpu.make_async_copy(
                    onesbuf, degs_sh.at[didx(w, b)], semS[b]).wait()

        dblk_start(0, 0)

        def deg_outer(m2, _):
            n = 2 * m2
            dblk_wait(0)
            dblk_start(1, n + 1)
            deg_proc(0)
            dblk_wait(1)
            dblk_start(0, n + 2)
            deg_proc(1)
            return 0
        lax.fori_loop(0, NGRP // 2, deg_outer, 0)
        dblk_wait(0)
        plsc.subcore_barrier()

        pltpu.sync_copy(degs_sh.at[pl.ds(row0, ROWS)], degbuf)

        def norm_body(i, _):
            d = degbuf[pl.ds(i * 16, 16)]
            y = 0.5 * (d + 1.0)
            for _ in range(16):
                y = 0.5 * (y + d / y)
            v = 1.0 / y
            for l in range(16):
                bc = jnp.broadcast_to(v[l], (16,))
                for jj in range(GW // 16):
                    nb[i * 16 + l, pl.ds(jj * 16, 16)] = bc
            return 0
        lax.fori_loop(0, ROWS // 16, norm_body, 0)

        def scale_rows(r, _):
            for jj in range(GW // 16):
                sl = pl.ds(jj * 16, 16)
                buf[r, sl] = buf[r, sl] * nb[r, sl]
            return 0

        for p in range(PASSES):
            grp = c * PASSES + p
            goff = grp * NPAD
            pltpu.sync_copy(x_hbm.at[grp, pl.ds(row0, ROWS)], buf)
            lax.fori_loop(0, ROWS, scale_rows, 0)
            pltpu.sync_copy(buf, g_hbm.at[pl.ds(goff + row0, ROWS)])
        plsc.subcore_barrier()

        def edge_proc(w):
            for b in range(NBUF):
                pltpu.make_async_copy(
                    g_hbm.at[sidx(w, b)], rowb[b], semG[b]).start()
            for b in range(NBUF):
                pltpu.make_async_copy(
                    g_hbm.at[sidx(w, b)], rowb[b], semG[b]).wait()
                pltpu.make_async_copy(
                    rowb[b], acc_sh.at[didx(w, b)], semS[b]).start(add=True)
            for b in range(NBUF):
                pltpu.make_async_copy(
                    rowb[b], acc_sh.at[didx(w, b)], semS[b]).wait()

        def hop(k, _):
            for p in range(PASSES):
                grp = c * PASSES + p
                goff = grp * NPAD
                soff = grp * E3

                sblk_start(0, 0, soff)
                dblk_start(0, 0)

                def edge_outer(m2, _2):
                    n = 2 * m2
                    sblk_wait(0)
                    dblk_wait(0)
                    sblk_start(1, n + 1, soff)
                    dblk_start(1, n + 1)
                    edge_proc(0)
                    sblk_wait(1)
                    dblk_wait(1)
                    sblk_start(0, n + 2, soff)
                    dblk_start(0, n + 2)
                    edge_proc(1)
                    return 0
                lax.fori_loop(0, NGRP // 2, edge_outer, 0)
                sblk_wait(0)
                dblk_wait(0)
                plsc.subcore_barrier()

                pltpu.sync_copy(acc_sh.at[pl.ds(row0, ROWS)], buf)
                lax.fori_loop(0, ROWS, scale_rows, 0)
                pltpu.sync_copy(buf, hs_hbm.at[k, grp, pl.ds(row0, ROWS)])
                lax.fori_loop(0, ROWS, scale_rows, 0)
                pltpu.sync_copy(buf, g_hbm.at[pl.ds(goff + row0, ROWS)])
                for m in range(ROWS // ZROWS):
                    pltpu.sync_copy(
                        zb, acc_sh.at[pl.ds(row0 + m * ZROWS, ZROWS)])
                plsc.subcore_barrier()
            return 0
        lax.fori_loop(0, K, hop, 0)

    return body(x4, srcflat, dstp, zeros2d, zeros1d)


RB = 1000


def _combine_body(x_ref, hs_ref, w_ref, o_ref):
    w = w_ref[...]
    xb = x_ref[...]
    sc = jnp.sum(xb * w, axis=1, keepdims=True)
    acc = xb / (1.0 + jnp.exp(-sc))
    for k in range(K):
        h = jnp.concatenate([hs_ref[k, g] for g in range(NG)], axis=1)
        sc = jnp.sum(h * w, axis=1, keepdims=True)
        acc = acc + h / (1.0 + jnp.exp(-sc))
    o_ref[...] = acc


def _combine(x, hs, W):
    return pl.pallas_call(
        _combine_body,
        grid=(N // RB,),
        in_specs=[
            pl.BlockSpec((RB, D), lambda i: (i, 0)),
            pl.BlockSpec((K, NG, RB, GW), lambda i: (0, 0, i, 0)),
            pl.BlockSpec((1, D), lambda i: (0, 0)),
        ],
        out_specs=pl.BlockSpec((RB, D), lambda i: (i, 0)),
        out_shape=jax.ShapeDtypeStruct((N, D), jnp.float32),
    )(x, hs, W)


def kernel(x, edge_index, W):
    src = edge_index[0]
    dst = edge_index[1]
    x4 = jnp.zeros((NG, NPAD, GW), jnp.float32)
    x4 = x4.at[:, :N].set(jnp.transpose(x.reshape(N, NG, GW), (1, 0, 2)))
    pad_rows = N + jnp.arange(PADC, dtype=jnp.int32) % (NPAD - N)
    srcp = jnp.concatenate([src, pad_rows])
    dstp = jnp.concatenate([dst, pad_rows])
    offs = (jnp.arange(NG, dtype=jnp.int32) * NPAD)[:, None]
    srcflat = (srcp[None, :] + offs).reshape(-1)
    zeros2d = jnp.zeros((ZROWS, GW), jnp.float32)
    zeros1d = jnp.zeros((ROWS,), jnp.float32)
    hs, _ = _sc_propagate(x4, srcflat, dstp, zeros2d, zeros1d)
    return _combine(x, hs, W)
